# Initial kernel scaffold; baseline (speedup 1.0000x reference)
import jax
import jax.numpy as jnp
from jax.experimental import pallas as pl


def kernel(idx, morphisms_logits):
    raise NotImplementedError("write your pallas kernel here")


# speedup vs baseline: 1.4037x; 1.4037x over previous
import functools

import jax
import jax.numpy as jnp
from jax import lax
from jax.experimental import pallas as pl
from jax.experimental.pallas import tpu as pltpu
from jax.experimental.pallas import tpu_sc as plsc

_V = 1000
_D = 1000
_B = 4096 * 20
_NC, _NS = 2, 16
_NW = _NC * _NS
_BPW = _B // _NW
_R = 64
_C = _BPW // _R


def _sigmoid_body(x_ref, o_ref):
    x = x_ref[...]
    o_ref[...] = 1.0 / (1.0 + jnp.exp(-x))


def _sigmoid_table(logits):
    return pl.pallas_call(
        _sigmoid_body,
        out_shape=jax.ShapeDtypeStruct(logits.shape, logits.dtype),
    )(logits)


_mesh = plsc.VectorSubcoreMesh(core_axis_name="c", subcore_axis_name="s")


@functools.partial(
    pl.kernel,
    out_type=jax.ShapeDtypeStruct((_B, _D), jnp.float32),
    mesh=_mesh,
    scratch_types=[
        pltpu.VMEM((_BPW,), jnp.int32),
        pltpu.VMEM((_R, _D), jnp.float32),
        pltpu.SemaphoreType.DMA,
    ],
    compiler_params=pltpu.CompilerParams(use_tc_tiling_on_sc=False),
)
def _gather_kernel(table_hbm, idx_hbm, out_hbm, idx_v, rows_v, sem):
    wid = lax.axis_index("s") * _NC + lax.axis_index("c")
    base = wid * _BPW
    pltpu.sync_copy(idx_hbm.at[pl.ds(base, _BPW)], idx_v)

    def chunk(c, carry):
        off = c * _R
        pltpu.async_copy(
            table_hbm.at[idx_v.at[pl.ds(off, _R)]], rows_v, sem
        ).wait()
        pltpu.sync_copy(rows_v, out_hbm.at[pl.ds(base + off, _R)])
        return carry

    lax.fori_loop(0, _C, chunk, 0)


def kernel(idx, morphisms_logits):
    table = _sigmoid_table(morphisms_logits)
    idx_flat = idx.reshape(-1).astype(jnp.int32)
    out = _gather_kernel(table, idx_flat)
    return out.reshape(idx.shape + (morphisms_logits.shape[0],))


# speedup vs baseline: 1.4377x; 1.0242x over previous
import functools

import jax
import jax.numpy as jnp
from jax import lax
from jax.experimental import pallas as pl
from jax.experimental.pallas import tpu as pltpu
from jax.experimental.pallas import tpu_sc as plsc

_V = 1000
_D = 1000
_B = 4096 * 20
_NC, _NS = 2, 16
_NW = _NC * _NS
_BPW = _B // _NW
_R = 64
_C = _BPW // _R


def _sigmoid_body(x_ref, o_ref):
    x = x_ref[...]
    o_ref[...] = 1.0 / (1.0 + jnp.exp(-x))


def _sigmoid_table(logits):
    return pl.pallas_call(
        _sigmoid_body,
        out_shape=jax.ShapeDtypeStruct(logits.shape, logits.dtype),
    )(logits)


_mesh = plsc.VectorSubcoreMesh(core_axis_name="c", subcore_axis_name="s")


@functools.partial(
    pl.kernel,
    out_type=jax.ShapeDtypeStruct((_B, _D), jnp.float32),
    mesh=_mesh,
    scratch_types=[
        pltpu.VMEM((_BPW,), jnp.int32),
        pltpu.VMEM((_R, _D), jnp.float32),
        pltpu.VMEM((_R, _D), jnp.float32),
        pltpu.SemaphoreType.DMA,
        pltpu.SemaphoreType.DMA,
        pltpu.SemaphoreType.DMA,
        pltpu.SemaphoreType.DMA,
    ],
    compiler_params=pltpu.CompilerParams(use_tc_tiling_on_sc=False),
)
def _gather_kernel(table_hbm, idx_hbm, out_hbm, idx_v, g0, g1,
                   gs0, gs1, os0, os1):
    sid = lax.axis_index("s")
    wid = sid * _NC + lax.axis_index("c")
    base = wid * _BPW
    pltpu.sync_copy(idx_hbm.at[pl.ds(base, _BPW)], idx_v)

    bufs = (g0, g1)
    gsems = (gs0, gs1)
    osems = (os0, os1)

    def gather_start(c, b):
        pltpu.make_async_copy(
            table_hbm.at[idx_v.at[pl.ds(c * _R, _R)]], bufs[b], gsems[b]
        ).start()

    def gather_wait(b):
        pltpu.make_async_copy(
            table_hbm.at[idx_v.at[pl.ds(0, _R)]], bufs[b], gsems[b]
        ).wait()

    def out_start(c, b):
        pltpu.make_async_copy(
            bufs[b], out_hbm.at[pl.ds(base + c * _R, _R)], osems[b]
        ).start()

    def out_wait(b):
        pltpu.make_async_copy(
            bufs[b], out_hbm.at[pl.ds(base, _R)], osems[b]
        ).wait()

    gather_start(0, 0)
    gather_start(1, 1)

    @pl.loop(0, _C, step=2)
    def _(c0):
        for b in range(2):
            c = c0 + b
            gather_wait(b)
            out_start(c, b)
            out_wait(b)

            @pl.when(c + 2 < _C)
            def _():
                gather_start(c + 2, b)


def kernel(idx, morphisms_logits):
    table = _sigmoid_table(morphisms_logits)
    idx_flat = idx.reshape(-1).astype(jnp.int32)
    out = _gather_kernel(table, idx_flat)
    return out.reshape(idx.shape + (morphisms_logits.shape[0],))
